# baseline (speedup 1.0000x reference)
import functools

import jax
import jax.numpy as jnp
from jax.experimental import pallas as pl
from jax.experimental.pallas import tpu as pltpu

B, T, C = 1, 2048, 768
H, D = 12, 64
K, P = 4, 2
ALPHA, BETA = 2.0, 32.0

TQ = 256


def _proj_kernel(x_ref, xT_ref, wqT_ref, wk_ref, wvT_ref, wdqT_ref, wdk_ref,
                 fq_ref, fk_ref, v_ref):
    x = x_ref[...]
    xT = xT_ref[...]
    qh = jnp.dot(x, wqT_ref[0], preferred_element_type=jnp.float32)
    kTh = jnp.dot(wk_ref[0], xT, preferred_element_type=jnp.float32)
    scale = jnp.float32(P - 1)
    beta = jnp.float32(BETA)
    c0 = jnp.exp(jnp.float32(-BETA / 2))
    dq = jax.nn.sigmoid(
        jnp.dot(qh, wdqT_ref[...], preferred_element_type=jnp.float32)) * scale
    dkT = jax.nn.sigmoid(
        jnp.dot(wdk_ref[...], kTh, preferred_element_type=jnp.float32)) * scale
    fq_ref[0] = jnp.concatenate(
        [jnp.exp(beta * dq), c0 * jnp.exp(-beta * dq)], axis=1)
    fk_ref[0] = jnp.concatenate(
        [jnp.exp(beta * dkT), c0 * jnp.exp(-beta * dkT)], axis=0)
    v_ref[0] = jnp.dot(x, wvT_ref[0], preferred_element_type=jnp.float32)


def _lcp_weights(fq, fk):
    one = jnp.float32(1.0)
    e = []
    for l in range(K):
        a = fq[:, l:l + 1]
        ia = fq[:, K + l:K + l + 1]
        b = fk[l:l + 1, :]
        ib = fk[K + l:K + l + 1, :]
        u = jnp.maximum(a * ib, ia * b)
        e.append(one + u)
    s2 = e[3]
    s1 = s2 * e[2]
    s0 = s1 * e[1]
    num = one + s2 + s1 + s0
    den = e[0] * s0
    return jnp.exp2(num / den)


def _attn_kernel(fq_ref, fk_ref, v_ref, woT_ref, y_ref):
    i = pl.program_id(0)
    h = pl.program_id(1)
    fq = fq_ref[0]

    def body(j, carry):
        acc, den = carry
        fk = fk_ref[0, :, pl.ds(j * TQ, TQ)]
        vblk = v_ref[0, pl.ds(j * TQ, TQ), :]
        w = _lcp_weights(fq, fk)
        acc = acc + jnp.dot(w, vblk, preferred_element_type=jnp.float32)
        den = den + jnp.sum(w, axis=1, keepdims=True)
        return acc, den

    acc0 = jnp.zeros((TQ, D), jnp.float32)
    den0 = jnp.zeros((TQ, 1), jnp.float32)
    acc, den = jax.lax.fori_loop(0, i, body, (acc0, den0))

    fk = fk_ref[0, :, pl.ds(i * TQ, TQ)]
    vblk = v_ref[0, pl.ds(i * TQ, TQ), :]
    w = _lcp_weights(fq, fk)
    rows = jax.lax.broadcasted_iota(jnp.int32, (TQ, TQ), 0)
    cols = jax.lax.broadcasted_iota(jnp.int32, (TQ, TQ), 1)
    w = jnp.where(cols <= rows, w, jnp.float32(0.0))
    acc = acc + jnp.dot(w, vblk, preferred_element_type=jnp.float32)
    den = den + jnp.sum(w, axis=1, keepdims=True)

    out = acc / den
    y = jnp.dot(out, woT_ref[...], preferred_element_type=jnp.float32)

    @pl.when(h == 0)
    def _():
        y_ref[...] = y

    @pl.when(h > 0)
    def _():
        y_ref[...] = y_ref[...] + y


@jax.jit
def _forward(x, Wq, Wk, Wv, Wo, Wdq, Wdk):
    x2 = x.reshape(T, C)
    fq, fk, v = pl.pallas_call(
        _proj_kernel,
        grid=(H,),
        in_specs=[
            pl.BlockSpec((T, C), lambda h: (0, 0)),
            pl.BlockSpec((C, T), lambda h: (0, 0)),
            pl.BlockSpec((1, C, D), lambda h: (h, 0, 0)),
            pl.BlockSpec((1, D, C), lambda h: (h, 0, 0)),
            pl.BlockSpec((1, C, D), lambda h: (h, 0, 0)),
            pl.BlockSpec((D, K), lambda h: (0, 0)),
            pl.BlockSpec((K, D), lambda h: (0, 0)),
        ],
        out_specs=(
            pl.BlockSpec((1, T, 2 * K), lambda h: (h, 0, 0)),
            pl.BlockSpec((1, 2 * K, T), lambda h: (h, 0, 0)),
            pl.BlockSpec((1, T, D), lambda h: (h, 0, 0)),
        ),
        out_shape=(
            jax.ShapeDtypeStruct((H, T, 2 * K), jnp.float32),
            jax.ShapeDtypeStruct((H, 2 * K, T), jnp.float32),
            jax.ShapeDtypeStruct((H, T, D), jnp.float32),
        ),
        compiler_params=pltpu.CompilerParams(
            dimension_semantics=("arbitrary",),
        ),
    )(x2, x2.T,
      Wq.T.reshape(C, H, D).transpose(1, 0, 2),
      Wk.reshape(H, D, C),
      Wv.T.reshape(C, H, D).transpose(1, 0, 2),
      Wdq.T, Wdk)

    nq = T // TQ
    y = pl.pallas_call(
        _attn_kernel,
        grid=(nq, H),
        in_specs=[
            pl.BlockSpec((1, TQ, 2 * K), lambda i, h: (h, i, 0)),
            pl.BlockSpec((1, 2 * K, T), lambda i, h: (h, 0, 0)),
            pl.BlockSpec((1, T, D), lambda i, h: (h, 0, 0)),
            pl.BlockSpec((D, C), lambda i, h: (h, 0)),
        ],
        out_specs=pl.BlockSpec((TQ, C), lambda i, h: (i, 0)),
        out_shape=jax.ShapeDtypeStruct((T, C), jnp.float32),
        compiler_params=pltpu.CompilerParams(
            dimension_semantics=("arbitrary", "arbitrary"),
        ),
    )(fq, fk, v, Wo.T)
    return y.reshape(B, T, C)


def kernel(x, Wq, Wk, Wv, Wo, Wdq, Wdk):
    return _forward(x, Wq, Wk, Wv, Wo, Wdq, Wdk)


# speedup vs baseline: 1.1086x; 1.1086x over previous
import functools

import jax
import jax.numpy as jnp
from jax.experimental import pallas as pl
from jax.experimental.pallas import tpu as pltpu

B, T, C = 1, 2048, 768
H, D = 12, 64
K, P = 4, 2
ALPHA, BETA = 2.0, 32.0

TQ = 256
VP = 128


def _proj_kernel(x_ref, wqT_ref, wkT_ref, wvT_ref, wdqT_ref, wdkT_ref,
                 dq_ref, dkT_ref, v_ref):
    x = x_ref[...]
    qh = jnp.dot(x, wqT_ref[0], preferred_element_type=jnp.float32)
    kh = jnp.dot(x, wkT_ref[0], preferred_element_type=jnp.float32)
    scale = jnp.float32(P - 1)
    dq = jax.nn.sigmoid(
        jnp.dot(qh, wdqT_ref[...], preferred_element_type=jnp.float32)) * scale
    dk = jax.nn.sigmoid(
        jnp.dot(kh, wdkT_ref[...], preferred_element_type=jnp.float32)) * scale
    dq_ref[0] = dq
    dkT_ref[0] = dk.T
    vh = jnp.dot(x, wvT_ref[0], preferred_element_type=jnp.float32)
    v_ref[0] = jnp.concatenate(
        [vh, jnp.ones((T, 1), jnp.float32), jnp.zeros((T, VP - D - 1), jnp.float32)],
        axis=1)


def _lcp_weights(dq, dkT):
    half = jnp.float32(0.5)
    beta = jnp.float32(BETA)
    cum = None
    lcp = None
    for l in range(K):
        a = dq[:, l:l + 1]
        b = dkT[l:l + 1, :]
        m = jax.nn.sigmoid(beta * (half - jnp.abs(a - b)))
        cum = m if cum is None else cum * m
        lcp = cum if lcp is None else lcp + cum
    return jnp.exp2(lcp)


def _attn_kernel(dq_ref, dkT_ref, v_ref, woT_ref, y_ref):
    i = pl.program_id(0)
    h = pl.program_id(1)
    dq = dq_ref[0]

    def body(j, acc):
        dkT = dkT_ref[0, :, pl.ds(j * TQ, TQ)]
        vblk = v_ref[0, pl.ds(j * TQ, TQ), :]
        w = _lcp_weights(dq, dkT)
        return acc + jnp.dot(w, vblk, preferred_element_type=jnp.float32)

    acc0 = jnp.zeros((TQ, VP), jnp.float32)
    acc = jax.lax.fori_loop(0, i, body, acc0)

    dkT = dkT_ref[0, :, pl.ds(i * TQ, TQ)]
    vblk = v_ref[0, pl.ds(i * TQ, TQ), :]
    w = _lcp_weights(dq, dkT)
    rows = jax.lax.broadcasted_iota(jnp.int32, (TQ, TQ), 0)
    cols = jax.lax.broadcasted_iota(jnp.int32, (TQ, TQ), 1)
    w = jnp.where(cols <= rows, w, jnp.float32(0.0))
    acc = acc + jnp.dot(w, vblk, preferred_element_type=jnp.float32)

    out = acc[:, :D] / acc[:, D:D + 1]
    y = jnp.dot(out, woT_ref[...], preferred_element_type=jnp.float32)

    @pl.when(h == 0)
    def _():
        y_ref[...] = y

    @pl.when(h > 0)
    def _():
        y_ref[...] = y_ref[...] + y


@jax.jit
def _forward(x, Wq, Wk, Wv, Wo, Wdq, Wdk):
    x2 = x.reshape(T, C)
    dq, dkT, v = pl.pallas_call(
        _proj_kernel,
        grid=(H,),
        in_specs=[
            pl.BlockSpec((T, C), lambda h: (0, 0)),
            pl.BlockSpec((1, C, D), lambda h: (h, 0, 0)),
            pl.BlockSpec((1, C, D), lambda h: (h, 0, 0)),
            pl.BlockSpec((1, C, D), lambda h: (h, 0, 0)),
            pl.BlockSpec((D, K), lambda h: (0, 0)),
            pl.BlockSpec((D, K), lambda h: (0, 0)),
        ],
        out_specs=(
            pl.BlockSpec((1, T, K), lambda h: (h, 0, 0)),
            pl.BlockSpec((1, K, T), lambda h: (h, 0, 0)),
            pl.BlockSpec((1, T, VP), lambda h: (h, 0, 0)),
        ),
        out_shape=(
            jax.ShapeDtypeStruct((H, T, K), jnp.float32),
            jax.ShapeDtypeStruct((H, K, T), jnp.float32),
            jax.ShapeDtypeStruct((H, T, VP), jnp.float32),
        ),
        compiler_params=pltpu.CompilerParams(
            dimension_semantics=("arbitrary",),
        ),
    )(x2,
      Wq.T.reshape(C, H, D).transpose(1, 0, 2),
      Wk.T.reshape(C, H, D).transpose(1, 0, 2),
      Wv.T.reshape(C, H, D).transpose(1, 0, 2),
      Wdq.T, Wdk.T)

    nq = T // TQ
    y = pl.pallas_call(
        _attn_kernel,
        grid=(nq, H),
        in_specs=[
            pl.BlockSpec((1, TQ, K), lambda i, h: (h, i, 0)),
            pl.BlockSpec((1, K, T), lambda i, h: (h, 0, 0)),
            pl.BlockSpec((1, T, VP), lambda i, h: (h, 0, 0)),
            pl.BlockSpec((D, C), lambda i, h: (h, 0)),
        ],
        out_specs=pl.BlockSpec((TQ, C), lambda i, h: (i, 0)),
        out_shape=jax.ShapeDtypeStruct((T, C), jnp.float32),
        compiler_params=pltpu.CompilerParams(
            dimension_semantics=("arbitrary", "arbitrary"),
        ),
    )(dq, dkT, v, Wo.T)
    return y.reshape(B, T, C)


def kernel(x, Wq, Wk, Wv, Wo, Wdq, Wdk):
    return _forward(x, Wq, Wk, Wv, Wo, Wdq, Wdk)
